# Initial kernel scaffold; baseline (speedup 1.0000x reference)
import jax
import jax.numpy as jnp
from jax.experimental import pallas as pl


def kernel(x):
    raise NotImplementedError("write your pallas kernel here")


# speedup vs baseline: 2.5592x; 2.5592x over previous
import functools

import jax
import jax.numpy as jnp
from jax import lax
from jax.experimental import pallas as pl
from jax.experimental.pallas import tpu as pltpu
from jax.experimental.pallas import tpu_sc as plsc

NC, NS, L = 2, 16, 16
NW = NC * NS
B, N = 128, 32768
ROWS_PER_W = B // NW
K = 20
KPAD = 32
NGROUPS = 16
GVECS = N // (L * NGROUPS)
BIG = 2**30


def _topk_one_row(row_v, gmax_v, gpos_v, lane):
    for g in range(NGROUPS):
        base = g * GVECS * L

        def p1(j, carry, base=base):
            m, pj = carry
            off = pl.multiple_of(base + j * L, L)
            v = row_v[pl.ds(off, L)]
            gt = v > m
            m = jnp.where(gt, v, m)
            pj = jnp.where(gt, jnp.broadcast_to(j, (L,)).astype(jnp.int32), pj)
            return m, pj

        m0 = row_v[pl.ds(base, L)]
        pj0 = jnp.zeros((L,), jnp.int32)
        m, pj = lax.fori_loop(1, GVECS, p1, (m0, pj0))
        gmax_v[pl.ds(g * L, L)] = m
        gpos_v[pl.ds(g * L, L)] = pj

    neg_v = jnp.full((L,), -jnp.inf, jnp.float32)

    def round_body(r, carry):
        ov0, ov1, oi0, oi1 = carry
        bm = gmax_v[pl.ds(0, L)]
        bg = jnp.zeros((L,), jnp.int32)
        bj = gpos_v[pl.ds(0, L)]
        for g in range(1, NGROUPS):
            v = gmax_v[pl.ds(g * L, L)]
            pj = gpos_v[pl.ds(g * L, L)]
            gt = v > bm
            bm = jnp.where(gt, v, bm)
            bg = jnp.where(gt, jnp.full((L,), g, jnp.int32), bg)
            bj = jnp.where(gt, pj, bj)
        mval = jnp.max(bm)
        pos = (bg * GVECS + bj) * L + lane
        pstar = jnp.min(jnp.where(bm == mval, pos, BIG))

        ov0 = jnp.where(lane == r, mval, ov0)
        ov1 = jnp.where(lane == r - L, mval, ov1)
        oi0 = jnp.where(lane == r, pstar, oi0)
        oi1 = jnp.where(lane == r - L, pstar, oi1)

        plsc.store_scatter(row_v, [jnp.broadcast_to(pstar, (L,))], neg_v,
                           mask=lane == 0)

        qstar = pstar // L
        lstar = pstar % L
        gstar = qstar // GVECS
        rbase = gstar * GVECS
        nm = None
        for t in range(GVECS // L):
            idx = (rbase + t * L + lane) * L + lstar
            v = plsc.load_gather(row_v, [idx])
            jj = t * L + lane
            if nm is None:
                nm, nj = v, jj
            else:
                gt = v > nm
                nm = jnp.where(gt, v, nm)
                nj = jnp.where(gt, jj, nj)
        nmax = jnp.max(nm)
        njstar = jnp.min(jnp.where(nm == nmax, nj, BIG))
        gidx = jnp.broadcast_to(gstar * L + lstar, (L,))
        plsc.store_scatter(gmax_v, [gidx], jnp.broadcast_to(nmax, (L,)),
                           mask=lane == 0)
        plsc.store_scatter(gpos_v, [gidx], jnp.broadcast_to(njstar, (L,)),
                           mask=lane == 0)
        return ov0, ov1, oi0, oi1

    zf = jnp.zeros((L,), jnp.float32)
    zi = jnp.zeros((L,), jnp.int32)
    return lax.fori_loop(0, K, round_body, (zf, zf, zi, zi))


@functools.partial(
    pl.kernel,
    out_type=(jax.ShapeDtypeStruct((B, KPAD), jnp.float32),
              jax.ShapeDtypeStruct((B, KPAD), jnp.int32)),
    mesh=plsc.VectorSubcoreMesh(core_axis_name="c", subcore_axis_name="s"),
    compiler_params=pltpu.CompilerParams(needs_layout_passes=False),
    scratch_types=[
        pltpu.VMEM((N,), jnp.float32),
        pltpu.VMEM((NGROUPS * L,), jnp.float32),
        pltpu.VMEM((NGROUPS * L,), jnp.int32),
        pltpu.VMEM((KPAD,), jnp.float32),
        pltpu.VMEM((KPAD,), jnp.int32),
    ],
)
def _sc_topk(x_hbm, outv_hbm, outi_hbm, row_v, gmax_v, gpos_v, outv_v, outi_v):
    wid = lax.axis_index("s") * NC + lax.axis_index("c")
    lane = lax.iota(jnp.int32, L)
    for rr in range(ROWS_PER_W):
        row = wid * ROWS_PER_W + rr
        pltpu.sync_copy(x_hbm.at[row], row_v)
        ov0, ov1, oi0, oi1 = _topk_one_row(row_v, gmax_v, gpos_v, lane)
        outv_v[pl.ds(0, L)] = ov0
        outv_v[pl.ds(L, L)] = ov1
        outi_v[pl.ds(0, L)] = oi0
        outi_v[pl.ds(L, L)] = oi1
        pltpu.sync_copy(outv_v, outv_hbm.at[row])
        pltpu.sync_copy(outi_v, outi_hbm.at[row])


def kernel(x):
    outv, outi = _sc_topk(x)
    return outv[:, :K], outi[:, :K]


# speedup vs baseline: 3.3281x; 1.3005x over previous
import functools

import jax
import jax.numpy as jnp
from jax import lax
from jax.experimental import pallas as pl
from jax.experimental.pallas import tpu as pltpu
from jax.experimental.pallas import tpu_sc as plsc

NC, NS, L = 2, 16, 16
NW = NC * NS
B, N = 128, 32768
ROWS_PER_W = B // NW
K = 20
KPAD = 32
NGROUPS = 16
GVECS = N // (L * NGROUPS)
BIG = 2**30


def _bucket_maxes(buf, gmax_v):
    for g in range(NGROUPS):
        base = g * GVECS * L

        def p1(i, carry, base=base):
            a0, a1, a2, a3 = carry
            off = pl.multiple_of(base + i * (4 * L), L)
            a0 = jnp.maximum(a0, buf[pl.ds(off, L)])
            a1 = jnp.maximum(a1, buf[pl.ds(off + L, L)])
            a2 = jnp.maximum(a2, buf[pl.ds(off + 2 * L, L)])
            a3 = jnp.maximum(a3, buf[pl.ds(off + 3 * L, L)])
            return a0, a1, a2, a3

        init = (buf[pl.ds(base, L)], buf[pl.ds(base + L, L)],
                buf[pl.ds(base + 2 * L, L)], buf[pl.ds(base + 3 * L, L)])
        a0, a1, a2, a3 = lax.fori_loop(1, GVECS // 4, p1, init, unroll=8)
        gmax_v[pl.ds(g * L, L)] = jnp.maximum(jnp.maximum(a0, a1),
                                              jnp.maximum(a2, a3))


def _topk_one_row(buf, gmax_v, lane):
    _bucket_maxes(buf, gmax_v)

    neg_v = jnp.full((L,), -jnp.inf, jnp.float32)

    def round_body(r, carry):
        ov0, ov1, oi0, oi1 = carry
        bm = gmax_v[pl.ds(0, L)]
        bg = jnp.zeros((L,), jnp.int32)
        for g in range(1, NGROUPS):
            v = gmax_v[pl.ds(g * L, L)]
            gt = v > bm
            bm = jnp.where(gt, v, bm)
            bg = jnp.where(gt, jnp.full((L,), g, jnp.int32), bg)
        mval = jnp.max(bm)
        tcnt = jnp.zeros((L,), jnp.int32)
        for g in range(NGROUPS):
            eqg = gmax_v[pl.ds(g * L, L)] == mval
            tcnt = tcnt + jnp.where(eqg, 1, 0)
        ttot = jnp.sum(tcnt)
        eq = bm == mval
        gstar = jnp.min(jnp.where(eq, bg, BIG))
        lstar = jnp.min(jnp.where(eq, lane, BIG))

        def fast(_):
            rbase = gstar * GVECS
            best = jnp.full((L,), BIG, jnp.int32)
            for t in range(GVECS // L):
                idx = (rbase + t * L + lane) * L + lstar
                v = plsc.load_gather(buf, [idx])
                jj = t * L + lane
                best = jnp.minimum(best, jnp.where(v == mval, jj, BIG))
            jstar = jnp.min(best)
            return (rbase + jstar) * L + lstar

        def slow(_):
            def sbody(i, bestc):
                off = pl.multiple_of(i * (4 * L), L)
                b0, b1, b2, b3 = bestc
                base_pos = i * (4 * L) + lane
                v0 = buf[pl.ds(off, L)]
                v1 = buf[pl.ds(off + L, L)]
                v2 = buf[pl.ds(off + 2 * L, L)]
                v3 = buf[pl.ds(off + 3 * L, L)]
                b0 = jnp.minimum(b0, jnp.where(v0 == mval, base_pos, BIG))
                b1 = jnp.minimum(b1, jnp.where(v1 == mval, base_pos + L, BIG))
                b2 = jnp.minimum(b2, jnp.where(v2 == mval, base_pos + 2 * L, BIG))
                b3 = jnp.minimum(b3, jnp.where(v3 == mval, base_pos + 3 * L, BIG))
                return b0, b1, b2, b3

            binit = tuple(jnp.full((L,), BIG, jnp.int32) for _ in range(4))
            b0, b1, b2, b3 = lax.fori_loop(0, N // (4 * L), sbody, binit,
                                           unroll=4)
            return jnp.min(jnp.minimum(jnp.minimum(b0, b1),
                                       jnp.minimum(b2, b3)))

        pstar = lax.cond(ttot == 1, fast, slow, 0)

        ov0 = jnp.where(lane == r, mval, ov0)
        ov1 = jnp.where(lane == r - L, mval, ov1)
        oi0 = jnp.where(lane == r, pstar, oi0)
        oi1 = jnp.where(lane == r - L, pstar, oi1)

        plsc.store_scatter(buf, [jnp.broadcast_to(pstar, (L,))], neg_v,
                           mask=lane == 0)
        qstar = pstar // L
        wl = pstar % L
        wg = qstar // GVECS
        rbase = wg * GVECS
        nm = None
        for t in range(GVECS // L):
            idx = (rbase + t * L + lane) * L + wl
            v = plsc.load_gather(buf, [idx])
            nm = v if nm is None else jnp.maximum(nm, v)
        nmax = jnp.max(nm)
        plsc.store_scatter(gmax_v, [jnp.broadcast_to(wg * L + wl, (L,))],
                           jnp.broadcast_to(nmax, (L,)), mask=lane == 0)
        return ov0, ov1, oi0, oi1

    zf = jnp.zeros((L,), jnp.float32)
    zi = jnp.zeros((L,), jnp.int32)
    return lax.fori_loop(0, K, round_body, (zf, zf, zi, zi))


@functools.partial(
    pl.kernel,
    out_type=(jax.ShapeDtypeStruct((B, KPAD), jnp.float32),
              jax.ShapeDtypeStruct((B, KPAD), jnp.int32)),
    mesh=plsc.VectorSubcoreMesh(core_axis_name="c", subcore_axis_name="s"),
    compiler_params=pltpu.CompilerParams(needs_layout_passes=False),
    scratch_types=[
        pltpu.VMEM((N,), jnp.float32),
        pltpu.VMEM((N,), jnp.float32),
        pltpu.VMEM((NGROUPS * L,), jnp.float32),
        pltpu.VMEM((KPAD,), jnp.float32),
        pltpu.VMEM((KPAD,), jnp.int32),
        pltpu.SemaphoreType.DMA,
        pltpu.SemaphoreType.DMA,
    ],
)
def _sc_topk(x_hbm, outv_hbm, outi_hbm, row_a, row_b, gmax_v, outv_v, outi_v,
             sem_a, sem_b):
    wid = lax.axis_index("s") * NC + lax.axis_index("c")
    lane = lax.iota(jnp.int32, L)
    base_row = wid * ROWS_PER_W
    bufs = (row_a, row_b)
    sems = (sem_a, sem_b)
    copies = [None, None]
    copies[0] = pltpu.async_copy(x_hbm.at[base_row], row_a, sem_a)
    for rr in range(ROWS_PER_W):
        buf = bufs[rr % 2]
        copies[rr % 2].wait()
        if rr + 1 < ROWS_PER_W:
            nxt = (rr + 1) % 2
            copies[nxt] = pltpu.async_copy(x_hbm.at[base_row + rr + 1],
                                           bufs[nxt], sems[nxt])
        ov0, ov1, oi0, oi1 = _topk_one_row(buf, gmax_v, lane)
        outv_v[pl.ds(0, L)] = ov0
        outv_v[pl.ds(L, L)] = ov1
        outi_v[pl.ds(0, L)] = oi0
        outi_v[pl.ds(L, L)] = oi1
        row = base_row + rr
        pltpu.sync_copy(outv_v, outv_hbm.at[row])
        pltpu.sync_copy(outi_v, outi_hbm.at[row])


def kernel(x):
    outv, outi = _sc_topk(x)
    return outv[:, :K], outi[:, :K]


# speedup vs baseline: 3.3323x; 1.0013x over previous
import functools

import jax
import jax.numpy as jnp
from jax import lax
from jax.experimental import pallas as pl
from jax.experimental.pallas import tpu as pltpu
from jax.experimental.pallas import tpu_sc as plsc

NC, NS, L = 2, 16, 16
NW = NC * NS
B, N = 128, 32768
ROWS_PER_W = B // NW
K = 20
KPAD = 32
NGROUPS = 16
GVECS = N // (L * NGROUPS)
BIG = 2**30


def _bucket_maxes(buf, gmax_v):
    for g in range(NGROUPS):
        base = g * GVECS * L

        def p1(i, carry, base=base):
            a0, a1, a2, a3 = carry
            off = pl.multiple_of(base + i * (4 * L), L)
            a0 = jnp.maximum(a0, buf[pl.ds(off, L)])
            a1 = jnp.maximum(a1, buf[pl.ds(off + L, L)])
            a2 = jnp.maximum(a2, buf[pl.ds(off + 2 * L, L)])
            a3 = jnp.maximum(a3, buf[pl.ds(off + 3 * L, L)])
            return a0, a1, a2, a3

        init = (buf[pl.ds(base, L)], buf[pl.ds(base + L, L)],
                buf[pl.ds(base + 2 * L, L)], buf[pl.ds(base + 3 * L, L)])
        a0, a1, a2, a3 = lax.fori_loop(1, GVECS // 4, p1, init, unroll=8)
        gmax_v[pl.ds(g * L, L)] = jnp.maximum(jnp.maximum(a0, a1),
                                              jnp.maximum(a2, a3))


def _round_one(buf, gmax_v, lane):
    bm = gmax_v[pl.ds(0, L)]
    bg = jnp.zeros((L,), jnp.int32)
    for g in range(1, NGROUPS):
        v = gmax_v[pl.ds(g * L, L)]
        gt = v > bm
        bm = jnp.where(gt, v, bm)
        bg = jnp.where(gt, jnp.full((L,), g, jnp.int32), bg)
    mval = jnp.max(bm)
    eq = bm == mval
    gstar = jnp.min(jnp.where(eq, bg, BIG))
    tie = eq & (bg == gstar)
    nl = plsc.all_reduce_population_count(tie)[0]
    lstar = jnp.min(jnp.where(tie, lane, BIG))
    rbase = gstar * GVECS

    def fast(_):
        best = jnp.full((L,), BIG, jnp.int32)
        vs = []
        for t in range(GVECS // L):
            idx = (rbase + t * L + lane) * L + lstar
            v = plsc.load_gather(buf, [idx])
            vs.append(v)
            jj = t * L + lane
            best = jnp.minimum(best, jnp.where(v == mval, jj, BIG))
        jstar = jnp.min(best)
        nm = None
        for t, v in enumerate(vs):
            jj = t * L + lane
            v2 = jnp.where(jj == jstar, -jnp.inf, v)
            nm = v2 if nm is None else jnp.maximum(nm, v2)
        return (rbase + jstar) * L + lstar, jnp.max(nm)

    def med(_):
        goff = rbase * L

        def sbody(i, bc):
            b0, b1 = bc
            off = pl.multiple_of(goff + i * (2 * L), L)
            p0 = goff + i * (2 * L) + lane
            v0 = buf[pl.ds(off, L)]
            v1 = buf[pl.ds(off + L, L)]
            b0 = jnp.minimum(b0, jnp.where(v0 == mval, p0, BIG))
            b1 = jnp.minimum(b1, jnp.where(v1 == mval, p0 + L, BIG))
            return b0, b1

        binit = (jnp.full((L,), BIG, jnp.int32),) * 2
        b0, b1 = lax.fori_loop(0, GVECS // 2, sbody, binit, unroll=4)
        pstar = jnp.min(jnp.minimum(b0, b1))
        wl = pstar % L
        nm = None
        for t in range(GVECS // L):
            idx = (rbase + t * L + lane) * L + wl
            v = plsc.load_gather(buf, [idx])
            v2 = jnp.where(idx == pstar, -jnp.inf, v)
            nm = v2 if nm is None else jnp.maximum(nm, v2)
        return pstar, jnp.max(nm)

    pstar, nmax = lax.cond(nl == 1, fast, med, 0)

    neg_v = jnp.full((L,), -jnp.inf, jnp.float32)
    plsc.store_scatter(buf, [jnp.broadcast_to(pstar, (L,))], neg_v,
                       mask=lane == 0)
    wl = pstar % L
    plsc.store_scatter(gmax_v, [jnp.broadcast_to(gstar * L + wl, (L,))],
                       jnp.broadcast_to(nmax, (L,)), mask=lane == 0)
    return mval, pstar


def _topk_pair(buf_x, buf_y, gmax_x, gmax_y, lane):
    _bucket_maxes(buf_x, gmax_x)
    _bucket_maxes(buf_y, gmax_y)

    def round_body(r, carry):
        xo0, xo1, xi0, xi1, yo0, yo1, yi0, yi1 = carry
        mvx, psx = _round_one(buf_x, gmax_x, lane)
        mvy, psy = _round_one(buf_y, gmax_y, lane)
        xo0 = jnp.where(lane == r, mvx, xo0)
        xo1 = jnp.where(lane == r - L, mvx, xo1)
        xi0 = jnp.where(lane == r, psx, xi0)
        xi1 = jnp.where(lane == r - L, psx, xi1)
        yo0 = jnp.where(lane == r, mvy, yo0)
        yo1 = jnp.where(lane == r - L, mvy, yo1)
        yi0 = jnp.where(lane == r, psy, yi0)
        yi1 = jnp.where(lane == r - L, psy, yi1)
        return xo0, xo1, xi0, xi1, yo0, yo1, yi0, yi1

    zf = jnp.zeros((L,), jnp.float32)
    zi = jnp.zeros((L,), jnp.int32)
    return lax.fori_loop(0, K, round_body, (zf, zf, zi, zi) * 2)


@functools.partial(
    pl.kernel,
    out_type=(jax.ShapeDtypeStruct((B, KPAD), jnp.float32),
              jax.ShapeDtypeStruct((B, KPAD), jnp.int32)),
    mesh=plsc.VectorSubcoreMesh(core_axis_name="c", subcore_axis_name="s"),
    compiler_params=pltpu.CompilerParams(needs_layout_passes=False),
    scratch_types=[
        pltpu.VMEM((N,), jnp.float32),
        pltpu.VMEM((N,), jnp.float32),
        pltpu.VMEM((N,), jnp.float32),
        pltpu.VMEM((NGROUPS * L,), jnp.float32),
        pltpu.VMEM((NGROUPS * L,), jnp.float32),
        pltpu.VMEM((KPAD,), jnp.float32),
        pltpu.VMEM((KPAD,), jnp.int32),
        pltpu.SemaphoreType.DMA,
        pltpu.SemaphoreType.DMA,
        pltpu.SemaphoreType.DMA,
    ],
)
def _sc_topk(x_hbm, outv_hbm, outi_hbm, row_a, row_b, row_c, gmax_x, gmax_y,
             outv_v, outi_v, sem_a, sem_b, sem_c):
    wid = lax.axis_index("s") * NC + lax.axis_index("c")
    lane = lax.iota(jnp.int32, L)
    base_row = wid * ROWS_PER_W

    cp_a = pltpu.async_copy(x_hbm.at[base_row], row_a, sem_a)
    cp_b = pltpu.async_copy(x_hbm.at[base_row + 1], row_b, sem_b)
    cp_a.wait()
    cp_b.wait()
    cp_c = pltpu.async_copy(x_hbm.at[base_row + 2], row_c, sem_c)

    def emit(row, ov0, ov1, oi0, oi1):
        outv_v[pl.ds(0, L)] = ov0
        outv_v[pl.ds(L, L)] = ov1
        outi_v[pl.ds(0, L)] = oi0
        outi_v[pl.ds(L, L)] = oi1
        pltpu.sync_copy(outv_v, outv_hbm.at[row])
        pltpu.sync_copy(outi_v, outi_hbm.at[row])

    r = _topk_pair(row_a, row_b, gmax_x, gmax_y, lane)
    cp_c.wait()
    cp_a2 = pltpu.async_copy(x_hbm.at[base_row + 3], row_a, sem_a)
    emit(base_row, *r[:4])
    emit(base_row + 1, *r[4:])
    cp_a2.wait()
    r = _topk_pair(row_c, row_a, gmax_x, gmax_y, lane)
    emit(base_row + 2, *r[:4])
    emit(base_row + 3, *r[4:])


def kernel(x):
    outv, outi = _sc_topk(x)
    return outv[:, :K], outi[:, :K]
